# Initial kernel scaffold; baseline (speedup 1.0000x reference)
import jax
import jax.numpy as jnp
from jax.experimental import pallas as pl


def kernel(hidden_states, router_weight, e_score_correction_bias, up_w, down_w, shared_up_w, shared_down_w):
    raise NotImplementedError("write your pallas kernel here")


# speedup vs baseline: 2.2728x; 2.2728x over previous
import functools

import jax
import jax.numpy as jnp
from jax import lax
from jax.experimental import pallas as pl
from jax.experimental.pallas import tpu as pltpu

T = 2048
H = 1024
E = 16
I = 512
IS = 1024
N_GROUP = 4
GROUP_SIZE = E // N_GROUP
TOPK_GROUP = 2
TOP_K = 2
SCALE = 2.5

_BT = 256
_NEG = -1e30


def _router_shared_body(x_ref, rwt_ref, bias_ref, sup_ref, sdn_ref,
                        ysh_ref, w_ref):
    x = x_ref[...]
    logits = jnp.dot(x, rwt_ref[...], preferred_element_type=jnp.float32)
    scores = jax.nn.sigmoid(logits)
    sfc = scores + bias_ref[...]

    cols = lax.broadcasted_iota(jnp.int32, (_BT, E), 1)
    grp = cols // GROUP_SIZE

    gscores = []
    for g in range(N_GROUP):
        vals = jnp.where(grp == g, sfc, _NEG)
        m1 = jnp.max(vals, axis=1, keepdims=True)
        i1 = jnp.min(jnp.where(vals == m1, cols, E + 1), axis=1, keepdims=True)
        vals2 = jnp.where(cols == i1, _NEG, vals)
        m2 = jnp.max(vals2, axis=1, keepdims=True)
        gscores.append(m1 + m2)

    gm1 = gscores[0]
    gi1 = jnp.zeros_like(gm1, dtype=jnp.int32)
    for g in range(1, N_GROUP):
        better = gscores[g] > gm1
        gi1 = jnp.where(better, g, gi1)
        gm1 = jnp.maximum(gscores[g], gm1)
    gm2 = jnp.full_like(gm1, _NEG)
    gi2 = jnp.zeros_like(gi1)
    for g in range(N_GROUP):
        cand = jnp.where(gi1 == g, _NEG, gscores[g])
        better = cand > gm2
        gi2 = jnp.where(better, g, gi2)
        gm2 = jnp.maximum(cand, gm2)

    group_mask = (grp == gi1) | (grp == gi2)
    msfc = jnp.where(group_mask, sfc, 0.0)

    m1 = jnp.max(msfc, axis=1, keepdims=True)
    e1 = jnp.min(jnp.where(msfc == m1, cols, E + 1), axis=1, keepdims=True)
    msfc2 = jnp.where(cols == e1, _NEG, msfc)
    m2 = jnp.max(msfc2, axis=1, keepdims=True)
    e2 = jnp.min(jnp.where(msfc2 == m2, cols, E + 1), axis=1, keepdims=True)

    sel1 = cols == e1
    sel2 = cols == e2
    w1 = jnp.sum(jnp.where(sel1, scores, 0.0), axis=1, keepdims=True)
    w2 = jnp.sum(jnp.where(sel2, scores, 0.0), axis=1, keepdims=True)
    denom = w1 + w2 + 1e-20
    w1 = w1 / denom * SCALE
    w2 = w2 / denom * SCALE
    w_ref[...] = jnp.where(sel1, w1, 0.0) + jnp.where(sel2, w2, 0.0)

    xb = x.astype(jnp.bfloat16)
    h = jnp.dot(xb, sup_ref[...], preferred_element_type=jnp.float32)
    r = jnp.maximum(h, 0.0)
    rr = (r * r).astype(jnp.bfloat16)
    ysh_ref[...] = jnp.dot(rr, sdn_ref[...], preferred_element_type=jnp.float32)


def _moe_dense_body(xb_ref, w_ref, up_ref, dn_ref, ysh_ref, out_ref):
    e = pl.program_id(0)
    up = up_ref[0].astype(jnp.bfloat16)
    dn = dn_ref[0].astype(jnp.bfloat16)
    cols = lax.broadcasted_iota(jnp.int32, (T, E), 1)
    wcol = jnp.sum(jnp.where(cols == e, w_ref[...], 0.0), axis=1,
                   keepdims=True)
    nchunk = 4
    ct = T // nchunk
    for tb in range(nchunk):
        sl = slice(tb * ct, (tb + 1) * ct)
        xc = xb_ref[sl, :]
        h = jnp.dot(xc, up, preferred_element_type=jnp.float32)
        r = jnp.maximum(h, 0.0)
        rr = (r * r).astype(jnp.bfloat16)
        yc = jnp.dot(rr, dn, preferred_element_type=jnp.float32)
        contrib = wcol[sl, :] * yc

        @pl.when(e == 0)
        def _():
            out_ref[sl, :] = ysh_ref[sl, :] + contrib

        @pl.when(e > 0)
        def _():
            out_ref[sl, :] = out_ref[sl, :] + contrib


@jax.jit
def kernel(hidden_states, router_weight, e_score_correction_bias, up_w,
           down_w, shared_up_w, shared_down_w):
    rwt = router_weight.T
    bias = e_score_correction_bias.reshape(1, E)
    sup = shared_up_w.astype(jnp.bfloat16)
    sdn = shared_down_w.astype(jnp.bfloat16)

    ysh, w = pl.pallas_call(
        _router_shared_body,
        grid=(T // _BT,),
        in_specs=[
            pl.BlockSpec((_BT, H), lambda i: (i, 0)),
            pl.BlockSpec((H, E), lambda i: (0, 0)),
            pl.BlockSpec((1, E), lambda i: (0, 0)),
            pl.BlockSpec((H, IS), lambda i: (0, 0)),
            pl.BlockSpec((IS, H), lambda i: (0, 0)),
        ],
        out_specs=[
            pl.BlockSpec((_BT, H), lambda i: (i, 0)),
            pl.BlockSpec((_BT, E), lambda i: (i, 0)),
        ],
        out_shape=[
            jax.ShapeDtypeStruct((T, H), jnp.float32),
            jax.ShapeDtypeStruct((T, E), jnp.float32),
        ],
    )(hidden_states, rwt, bias, sup, sdn)

    xb = hidden_states.astype(jnp.bfloat16)
    y = pl.pallas_call(
        _moe_dense_body,
        grid=(E,),
        in_specs=[
            pl.BlockSpec((T, H), lambda e: (0, 0)),
            pl.BlockSpec((T, E), lambda e: (0, 0)),
            pl.BlockSpec((1, H, I), lambda e: (e, 0, 0)),
            pl.BlockSpec((1, I, H), lambda e: (e, 0, 0)),
            pl.BlockSpec((T, H), lambda e: (0, 0)),
        ],
        out_specs=pl.BlockSpec((T, H), lambda e: (0, 0)),
        out_shape=jax.ShapeDtypeStruct((T, H), jnp.float32),
        compiler_params=pltpu.CompilerParams(
            dimension_semantics=("arbitrary",)),
    )(xb, w, up_w, down_w, ysh)
    return y
